# baseline (speedup 1.0000x reference)
import functools

import jax
import jax.numpy as jnp
from jax import lax
from jax.experimental import pallas as pl
from jax.experimental.pallas import tpu as pltpu
from jax.experimental.pallas import tpu_sc as plsc

E = 8
D = 1024
F = 4096
S = 2048
TM = 256
NT = S // TM + E
ROWS = NT * TM
FT = 512
NF = F // FT
EPS = 1e-12
CHUNK = 128

NW = 32
TPW = S // NW


def _route_body(ids_ref, dest_ref, off_ref, nt_ref, rank_ref):
    tril = (lax.broadcasted_iota(jnp.int32, (CHUNK, CHUNK), 0)
            >= lax.broadcasted_iota(jnp.int32, (CHUNK, CHUNK), 1)
            ).astype(jnp.float32)
    eids = lax.broadcasted_iota(jnp.int32, (CHUNK, E), 1)

    def pass1(c, carry):
        ids = ids_ref[pl.ds(c * CHUNK, CHUNK), :]
        oh = (ids == eids).astype(jnp.float32)
        incl = jnp.dot(tril, oh, preferred_element_type=jnp.float32) + carry
        rank_ref[pl.ds(c * CHUNK, CHUNK), :] = (
            jnp.sum(oh * incl, axis=1, keepdims=True) - 1.0)
        return carry + jnp.sum(oh, axis=0, keepdims=True)

    counts = lax.fori_loop(0, S // CHUNK, pass1,
                           jnp.zeros((1, E), jnp.float32))
    ntiles = (counts.astype(jnp.int32) + (TM - 1)) // TM
    strict_lt = (lax.broadcasted_iota(jnp.int32, (E, E), 0)
                 < lax.broadcasted_iota(jnp.int32, (E, E), 1)
                 ).astype(jnp.float32)
    off_tiles = jnp.dot(ntiles.astype(jnp.float32), strict_lt,
                        preferred_element_type=jnp.float32)
    off_rows = off_tiles * float(TM)
    nt_ref[...] = ntiles
    off_ref[...] = off_rows.astype(jnp.int32)

    def pass2(c, _):
        ids = ids_ref[pl.ds(c * CHUNK, CHUNK), :]
        oh = (ids == eids).astype(jnp.float32)
        base = jnp.sum(oh * off_rows, axis=1, keepdims=True)
        dest_ref[pl.ds(c * CHUNK, CHUNK), :] = (
            base + rank_ref[pl.ds(c * CHUNK, CHUNK), :]).astype(jnp.int32)
        return 0

    lax.fori_loop(0, S // CHUNK, pass2, 0)


def _route(ids2d):
    return pl.pallas_call(
        _route_body,
        out_shape=(
            jax.ShapeDtypeStruct((S, 1), jnp.int32),
            jax.ShapeDtypeStruct((1, E), jnp.int32),
            jax.ShapeDtypeStruct((1, E), jnp.int32),
        ),
        scratch_shapes=[pltpu.VMEM((S, 1), jnp.float32)],
    )(ids2d)


@functools.lru_cache(maxsize=None)
def _sc_kernels():
    mesh = plsc.VectorSubcoreMesh(core_axis_name="c", subcore_axis_name="s")

    @functools.partial(
        pl.kernel,
        out_type=jax.ShapeDtypeStruct((ROWS, D), jnp.float32),
        mesh=mesh,
        scratch_types=[
            pltpu.VMEM((TPW,), jnp.int32),
            pltpu.VMEM((TPW, D), jnp.float32),
            pltpu.SemaphoreType.DMA,
        ],
    )
    def sc_scatter(x_hbm, dest_hbm, out_hbm, idx_v, rows_v, sem):
        wid = lax.axis_index("s") * 2 + lax.axis_index("c")
        base = wid * TPW
        pltpu.sync_copy(dest_hbm.at[pl.ds(base, TPW)], idx_v)
        pltpu.sync_copy(x_hbm.at[pl.ds(base, TPW)], rows_v)
        pltpu.async_copy(rows_v, out_hbm.at[idx_v], sem).wait()

    @functools.partial(
        pl.kernel,
        out_type=jax.ShapeDtypeStruct((S, D), jnp.float32),
        mesh=mesh,
        scratch_types=[
            pltpu.VMEM((TPW,), jnp.int32),
            pltpu.VMEM((TPW, D), jnp.float32),
            pltpu.SemaphoreType.DMA,
        ],
    )
    def sc_gather(y_hbm, dest_hbm, out_hbm, idx_v, rows_v, sem):
        wid = lax.axis_index("s") * 2 + lax.axis_index("c")
        base = wid * TPW
        pltpu.sync_copy(dest_hbm.at[pl.ds(base, TPW)], idx_v)
        pltpu.async_copy(y_hbm.at[idx_v], rows_v, sem).wait()
        pltpu.sync_copy(rows_v, out_hbm.at[pl.ds(base, TPW)])

    return sc_scatter, sc_gather


def _ffn_body(off_ref, nt_ref, x_ref, w1_ref, b1_ref, w2_ref, b2_ref,
              g_ref, bt_ref, out_ref, xb_ref, w1b_ref, w2b_ref):
    e = pl.program_id(0)
    f = pl.program_id(1)

    @pl.when(jnp.logical_and(e == 0, f == 0))
    def _():
        xb_ref[...] = x_ref[...].astype(jnp.bfloat16)

    w1b_ref[...] = w1_ref[0].astype(jnp.bfloat16)
    w2b_ref[...] = w2_ref[0].astype(jnp.bfloat16)
    base = off_ref[0, e]
    nt = nt_ref[0, e]

    def tile_body(t, _):
        row = pl.multiple_of(base + t * TM, TM)
        xt = xb_ref[pl.ds(row, TM), :]
        h = jnp.dot(xt, w1b_ref[...],
                    preferred_element_type=jnp.float32) + b1_ref[0]
        h = 0.5 * h * (1.0 + lax.erf(h * 0.7071067811865476))
        contrib = jnp.dot(h.astype(jnp.bfloat16), w2b_ref[...],
                          preferred_element_type=jnp.float32)

        @pl.when(f == 0)
        def _():
            out_ref[pl.ds(row, TM), :] = contrib + b2_ref[0]

        @pl.when(jnp.logical_and(f > 0, f < NF - 1))
        def _():
            out_ref[pl.ds(row, TM), :] += contrib

        @pl.when(f == NF - 1)
        def _():
            y = (out_ref[pl.ds(row, TM), :] + contrib
                 + x_ref[pl.ds(row, TM), :])
            mean = jnp.mean(y, axis=1, keepdims=True)
            yc = y - mean
            var = jnp.mean(yc * yc, axis=1, keepdims=True)
            yn = yc * lax.rsqrt(var + EPS)
            out_ref[pl.ds(row, TM), :] = yn * g_ref[...] + bt_ref[...]

        return 0

    lax.fori_loop(0, nt, tile_body, 0)


def _ffn(off, ntiles, x_pad, W1, b1, W2, b2, gamma2, beta2):
    grid_spec = pltpu.PrefetchScalarGridSpec(
        num_scalar_prefetch=2,
        grid=(E, NF),
        in_specs=[
            pl.BlockSpec((ROWS, D), lambda e, f, off, nt: (0, 0)),
            pl.BlockSpec((1, D, FT), lambda e, f, off, nt: (e, 0, f)),
            pl.BlockSpec((1, 1, FT), lambda e, f, off, nt: (e, 0, f)),
            pl.BlockSpec((1, FT, D), lambda e, f, off, nt: (e, f, 0)),
            pl.BlockSpec((1, 1, D), lambda e, f, off, nt: (e, 0, 0)),
            pl.BlockSpec((1, D), lambda e, f, off, nt: (0, 0)),
            pl.BlockSpec((1, D), lambda e, f, off, nt: (0, 0)),
        ],
        out_specs=pl.BlockSpec((ROWS, D), lambda e, f, off, nt: (0, 0)),
        scratch_shapes=[
            pltpu.VMEM((ROWS, D), jnp.bfloat16),
            pltpu.VMEM((D, FT), jnp.bfloat16),
            pltpu.VMEM((FT, D), jnp.bfloat16),
        ],
    )
    return pl.pallas_call(
        _ffn_body,
        grid_spec=grid_spec,
        out_shape=jax.ShapeDtypeStruct((ROWS, D), jnp.float32),
        compiler_params=pltpu.CompilerParams(
            dimension_semantics=("arbitrary", "arbitrary")),
    )(off, ntiles, x_pad, W1, b1.reshape(E, 1, F), W2, b2.reshape(E, 1, D),
      gamma2, beta2)


def kernel(hidden_states, expert_ids, W1, b1, W2, b2, gamma, beta):
    Bh, Sh, Dh = hidden_states.shape
    x = hidden_states.reshape(Sh, Dh)
    ids2d = expert_ids.astype(jnp.int32).reshape(Sh, 1)
    dest2d, off2d, nt2d = _route(ids2d)
    dest = dest2d.reshape(Sh)
    sc_scatter, sc_gather = _sc_kernels()
    x_pad = sc_scatter(x, dest)
    y_pad = _ffn(off2d, nt2d, x_pad, W1, b1, W2, b2,
                 gamma.reshape(1, Dh), beta.reshape(1, Dh))
    y = sc_gather(y_pad, dest)
    return y.reshape(Bh, Sh, Dh)


# speedup vs baseline: 1.2132x; 1.2132x over previous
import functools

import jax
import jax.numpy as jnp
from jax import lax
from jax.experimental import pallas as pl
from jax.experimental.pallas import tpu as pltpu
from jax.experimental.pallas import tpu_sc as plsc

E = 8
D = 1024
F = 4096
S = 2048
TM = 128
NT = S // TM + E
ROWS = NT * TM
FT = 1024
NF = F // FT
EPS = 1e-12
CHUNK = 128

NW = 32
TPW = S // NW


def _route_body(ids_ref, dest_ref, off_ref, nt_ref, rank_ref):
    tril = (lax.broadcasted_iota(jnp.int32, (CHUNK, CHUNK), 0)
            >= lax.broadcasted_iota(jnp.int32, (CHUNK, CHUNK), 1)
            ).astype(jnp.float32)
    eids = lax.broadcasted_iota(jnp.int32, (CHUNK, E), 1)

    def pass1(c, carry):
        ids = ids_ref[pl.ds(c * CHUNK, CHUNK), :]
        oh = (ids == eids).astype(jnp.float32)
        incl = jnp.dot(tril, oh, preferred_element_type=jnp.float32) + carry
        rank_ref[pl.ds(c * CHUNK, CHUNK), :] = (
            jnp.sum(oh * incl, axis=1, keepdims=True) - 1.0)
        return carry + jnp.sum(oh, axis=0, keepdims=True)

    counts = lax.fori_loop(0, S // CHUNK, pass1,
                           jnp.zeros((1, E), jnp.float32))
    ntiles = (counts.astype(jnp.int32) + (TM - 1)) // TM
    strict_lt = (lax.broadcasted_iota(jnp.int32, (E, E), 0)
                 < lax.broadcasted_iota(jnp.int32, (E, E), 1)
                 ).astype(jnp.float32)
    off_tiles = jnp.dot(ntiles.astype(jnp.float32), strict_lt,
                        preferred_element_type=jnp.float32)
    off_rows = off_tiles * float(TM)
    nt_ref[...] = ntiles
    off_ref[...] = off_rows.astype(jnp.int32)

    def pass2(c, _):
        ids = ids_ref[pl.ds(c * CHUNK, CHUNK), :]
        oh = (ids == eids).astype(jnp.float32)
        base = jnp.sum(oh * off_rows, axis=1, keepdims=True)
        dest_ref[pl.ds(c * CHUNK, CHUNK), :] = (
            base + rank_ref[pl.ds(c * CHUNK, CHUNK), :]).astype(jnp.int32)
        return 0

    lax.fori_loop(0, S // CHUNK, pass2, 0)


def _route(ids2d):
    return pl.pallas_call(
        _route_body,
        out_shape=(
            jax.ShapeDtypeStruct((S, 1), jnp.int32),
            jax.ShapeDtypeStruct((1, E), jnp.int32),
            jax.ShapeDtypeStruct((1, E), jnp.int32),
        ),
        scratch_shapes=[pltpu.VMEM((S, 1), jnp.float32)],
    )(ids2d)


@functools.lru_cache(maxsize=None)
def _sc_kernels():
    mesh = plsc.VectorSubcoreMesh(core_axis_name="c", subcore_axis_name="s")

    @functools.partial(
        pl.kernel,
        out_type=jax.ShapeDtypeStruct((ROWS, D), jnp.float32),
        mesh=mesh,
        scratch_types=[
            pltpu.VMEM((TPW,), jnp.int32),
            pltpu.VMEM((TPW, D), jnp.float32),
            pltpu.SemaphoreType.DMA,
        ],
    )
    def sc_scatter(x_hbm, dest_hbm, out_hbm, idx_v, rows_v, sem):
        wid = lax.axis_index("s") * 2 + lax.axis_index("c")
        base = wid * TPW
        pltpu.sync_copy(dest_hbm.at[pl.ds(base, TPW)], idx_v)
        pltpu.sync_copy(x_hbm.at[pl.ds(base, TPW)], rows_v)
        pltpu.async_copy(rows_v, out_hbm.at[idx_v], sem).wait()

    @functools.partial(
        pl.kernel,
        out_type=jax.ShapeDtypeStruct((S, D), jnp.float32),
        mesh=mesh,
        scratch_types=[
            pltpu.VMEM((TPW,), jnp.int32),
            pltpu.VMEM((TPW, D), jnp.float32),
            pltpu.SemaphoreType.DMA,
        ],
    )
    def sc_gather(y_hbm, dest_hbm, out_hbm, idx_v, rows_v, sem):
        wid = lax.axis_index("s") * 2 + lax.axis_index("c")
        base = wid * TPW
        pltpu.sync_copy(dest_hbm.at[pl.ds(base, TPW)], idx_v)
        pltpu.async_copy(y_hbm.at[idx_v], rows_v, sem).wait()
        pltpu.sync_copy(rows_v, out_hbm.at[pl.ds(base, TPW)])

    return sc_scatter, sc_gather


def _ffn_body(off_ref, nt_ref, x_ref, w1_ref, b1_ref, w2_ref, b2_ref,
              g_ref, bt_ref, out_ref):
    e = pl.program_id(0)
    f = pl.program_id(1)
    w1 = w1_ref[0]
    w2 = w2_ref[0]
    base = off_ref[0, e]
    nt = nt_ref[0, e]

    def tile_body(t, _):
        row = pl.multiple_of(base + t * TM, TM)
        xt = x_ref[pl.ds(row, TM), :]
        h = jnp.dot(xt, w1, preferred_element_type=jnp.float32) + b1_ref[0]
        h = 0.5 * h * (1.0 + lax.erf(h * 0.7071067811865476))
        contrib = jnp.dot(h, w2, preferred_element_type=jnp.float32)

        @pl.when(f == 0)
        def _():
            out_ref[pl.ds(row, TM), :] = contrib + b2_ref[0]

        @pl.when(jnp.logical_and(f > 0, f < NF - 1))
        def _():
            out_ref[pl.ds(row, TM), :] += contrib

        @pl.when(f == NF - 1)
        def _():
            y = out_ref[pl.ds(row, TM), :] + contrib + xt
            mean = jnp.mean(y, axis=1, keepdims=True)
            yc = y - mean
            var = jnp.mean(yc * yc, axis=1, keepdims=True)
            yn = yc * lax.rsqrt(var + EPS)
            out_ref[pl.ds(row, TM), :] = yn * g_ref[...] + bt_ref[...]

        return 0

    lax.fori_loop(0, nt, tile_body, 0)


def _ffn(off, ntiles, x_pad, W1, b1, W2, b2, gamma2, beta2):
    grid_spec = pltpu.PrefetchScalarGridSpec(
        num_scalar_prefetch=2,
        grid=(E, NF),
        in_specs=[
            pl.BlockSpec((ROWS, D), lambda e, f, off, nt: (0, 0)),
            pl.BlockSpec((1, D, FT), lambda e, f, off, nt: (e, 0, f)),
            pl.BlockSpec((1, 1, FT), lambda e, f, off, nt: (e, 0, f)),
            pl.BlockSpec((1, FT, D), lambda e, f, off, nt: (e, f, 0)),
            pl.BlockSpec((1, 1, D), lambda e, f, off, nt: (e, 0, 0)),
            pl.BlockSpec((1, D), lambda e, f, off, nt: (0, 0)),
            pl.BlockSpec((1, D), lambda e, f, off, nt: (0, 0)),
        ],
        out_specs=pl.BlockSpec((ROWS, D), lambda e, f, off, nt: (0, 0)),
    )
    return pl.pallas_call(
        _ffn_body,
        grid_spec=grid_spec,
        out_shape=jax.ShapeDtypeStruct((ROWS, D), jnp.float32),
        compiler_params=pltpu.CompilerParams(
            dimension_semantics=("arbitrary", "arbitrary")),
    )(off, ntiles, x_pad, W1, b1.reshape(E, 1, F), W2, b2.reshape(E, 1, D),
      gamma2, beta2)


def kernel(hidden_states, expert_ids, W1, b1, W2, b2, gamma, beta):
    Bh, Sh, Dh = hidden_states.shape
    x = hidden_states.reshape(Sh, Dh)
    ids2d = expert_ids.astype(jnp.int32).reshape(Sh, 1)
    dest2d, off2d, nt2d = _route(ids2d)
    dest = dest2d.reshape(Sh)
    sc_scatter, sc_gather = _sc_kernels()
    x_pad = sc_scatter(x, dest)
    y_pad = _ffn(off2d, nt2d, x_pad, W1, b1, W2, b2,
                 gamma.reshape(1, Dh), beta.reshape(1, Dh))
    y = sc_gather(y_pad, dest)
    return y.reshape(Bh, Sh, Dh)
